# baseline (speedup 1.0000x reference)
import jax
import jax.numpy as jnp
from jax.experimental import pallas as pl
from jax.experimental.pallas import tpu as pltpu

B, T, D = 8, 2048, 256
K = 1024
SPAN = 8
S = T // SPAN

_HI = jax.lax.Precision.HIGHEST


def _vq_body(x_ref, emb_ref, wb_ref, bb_ref,
             q_ref, idx_ref, bnd_ref, loss_ref, acc_ref):
    b = pl.program_id(0)

    xb = x_ref[0]
    wb = wb_ref[...]
    bb = bb_ref[0, 0]
    emb = emb_ref[...]

    wb8 = jnp.broadcast_to(wb, (8, D))
    logits8 = jax.lax.dot_general(xb, wb8,
                                  (((1,), (1,)), ((), ()))) + bb
    bound = (logits8[:, 0:1] > 0.0).astype(jnp.float32)
    bnd_ref[0] = bound

    t_of = jax.lax.broadcasted_iota(jnp.int32, (S, T), 1)
    s_of = jax.lax.broadcasted_iota(jnp.int32, (S, T), 0)
    pool_mat = jnp.where(t_of // SPAN == s_of, 1.0 / SPAN, 0.0)
    pooled = jax.lax.dot_general(pool_mat, xb,
                                 (((1,), (0,)), ((), ())),
                                 precision=_HI)

    ones_d8 = jnp.full((D, 8), 1.0, jnp.float32)
    en_col = jax.lax.dot_general(emb * emb, ones_d8,
                                 (((1,), (0,)), ((), ())),
                                 precision=_HI)[:, 0:1]

    dots_t = jax.lax.dot_general(emb, pooled,
                                 (((1,), (1,)), ((), ())))
    dist_t = en_col - 2.0 * dots_t

    d = dist_t
    i = jax.lax.broadcasted_iota(jnp.int32, (K, S), 0)
    w = K // 2
    while w >= 1:
        d0, d1 = d[:w, :], d[w:2 * w, :]
        i0, i1 = i[:w, :], i[w:2 * w, :]
        take = d1 < d0
        d = jnp.where(take, d1, d0)
        i = jnp.where(take, i1, i0)
        w //= 2
    idx_row = i

    k_col = jax.lax.broadcasted_iota(jnp.int32, (K, S), 0)
    onehot_t = (k_col == idx_row).astype(jnp.float32)
    quantized = jax.lax.dot_general(onehot_t, emb,
                                    (((0,), (0,)), ((), ())),
                                    precision=_HI)

    k_vals8 = jnp.broadcast_to(
        jax.lax.broadcasted_iota(jnp.int32, (K, 1), 0).astype(jnp.float32),
        (K, 8))
    idxc = jax.lax.dot_general(onehot_t, k_vals8,
                               (((0,), (0,)), ((), ())),
                               precision=_HI)[:, 0:1].astype(jnp.int32)
    idx_ref[0] = jnp.broadcast_to(idxc, (S, SPAN))

    s_of2 = jax.lax.broadcasted_iota(jnp.int32, (T, S), 1)
    t_of2 = jax.lax.broadcasted_iota(jnp.int32, (T, S), 0)
    expand_mat = (t_of2 // SPAN == s_of2).astype(jnp.float32)
    q_ref[0] = jax.lax.dot_general(expand_mat, quantized,
                                   (((1,), (0,)), ((), ())),
                                   precision=_HI)

    diff2 = (quantized - pooled) ** 2
    ones_8s = jnp.full((8, S), 1.0, jnp.float32)
    row8 = jax.lax.dot_general(ones_8s, diff2,
                               (((1,), (0,)), ((), ())),
                               precision=_HI)
    sq11 = jax.lax.dot_general(row8, ones_d8,
                               (((1,), (0,)), ((), ())),
                               precision=_HI)[0:1, 0:1]
    ones_8t = jnp.full((8, T), 1.0, jnp.float32)
    bound8 = jnp.broadcast_to(bound, (T, 8))
    sb11 = jax.lax.dot_general(ones_8t, bound8,
                               (((1,), (0,)), ((), ())),
                               precision=_HI)[0:1, 0:1]

    @pl.when(b == 0)
    def _():
        acc_ref[:, 0:1] = sq11
        acc_ref[:, 1:2] = sb11

    @pl.when(b > 0)
    def _():
        acc_ref[:, 0:1] += sq11
        acc_ref[:, 1:2] += sb11

    @pl.when(b == B - 1)
    def _():
        e_latent = acc_ref[:, 0:1] * (1.0 / (B * S * D))
        rate = acc_ref[:, 1:2] * (1.0 / (B * T))
        loss_ref[...] = 0.25 * e_latent + 0.01 * (rate - 1.0 / SPAN) ** 2


@jax.jit
def kernel(x, embedding, Wb, bb):
    wb2 = Wb.reshape(1, D)
    bb2 = jnp.asarray(bb, jnp.float32).reshape(1, 1)

    q, idx3, bnd, loss = pl.pallas_call(
        _vq_body,
        grid=(B,),
        in_specs=[
            pl.BlockSpec((1, T, D), lambda b: (b, 0, 0)),
            pl.BlockSpec((K, D), lambda b: (0, 0)),
            pl.BlockSpec((1, D), lambda b: (0, 0)),
            pl.BlockSpec((1, 1), lambda b: (0, 0), memory_space=pltpu.SMEM),
        ],
        out_specs=[
            pl.BlockSpec((1, T, D), lambda b: (b, 0, 0)),
            pl.BlockSpec((1, S, SPAN), lambda b: (b, 0, 0)),
            pl.BlockSpec((1, T, 1), lambda b: (b, 0, 0)),
            pl.BlockSpec((1, 1), lambda b: (0, 0)),
        ],
        out_shape=[
            jax.ShapeDtypeStruct((B, T, D), jnp.float32),
            jax.ShapeDtypeStruct((B, S, SPAN), jnp.int32),
            jax.ShapeDtypeStruct((B, T, 1), jnp.float32),
            jax.ShapeDtypeStruct((1, 1), jnp.float32),
        ],
        scratch_shapes=[pltpu.VMEM((1, 2), jnp.float32)],
    )(x, embedding, wb2, bb2)

    indices_out = idx3.reshape(B, T)
    total_loss = loss[0, 0]
    return q, total_loss, indices_out, bnd.reshape(B, T)


# speedup vs baseline: 2.0057x; 2.0057x over previous
import jax
import jax.numpy as jnp
from jax.experimental import pallas as pl
from jax.experimental.pallas import tpu as pltpu

B, T, D = 8, 2048, 256
K = 1024
SPAN = 8
S = T // SPAN

_HI = jax.lax.Precision.HIGHEST


def _vq_body(x_ref, emb_ref, wb_ref, bb_ref,
             q_ref, idx_ref, bnd_ref, loss_ref, acc_ref, en_ref):
    b = pl.program_id(0)

    x4 = x_ref[0]
    xb = x4.reshape(T, D)
    wb = wb_ref[...]
    bb = bb_ref[0, 0]
    emb = emb_ref[...]

    wb8 = jnp.broadcast_to(wb, (8, D))
    logits8 = jax.lax.dot_general(xb, wb8,
                                  (((1,), (1,)), ((), ()))) + bb
    bound = (logits8[:, 0:1] > 0.0).astype(jnp.float32)
    bnd_ref[0] = bound

    acc = x4[:, 0, :]
    for j in range(1, SPAN):
        acc = acc + x4[:, j, :]
    pooled = acc * (1.0 / SPAN)

    @pl.when(b == 0)
    def _():
        ones_d8 = jnp.full((D, 8), 1.0, jnp.float32)
        en_ref[...] = jax.lax.dot_general(emb * emb, ones_d8,
                                          (((1,), (0,)), ((), ())),
                                          precision=_HI)

    en_col = en_ref[:, 0:1]

    dots_t = jax.lax.dot_general(emb, pooled,
                                 (((1,), (1,)), ((), ())))
    dist_t = en_col - 2.0 * dots_t

    d = dist_t
    i = jax.lax.broadcasted_iota(jnp.int32, (K, S), 0)
    w = K // 2
    while w >= 1:
        d0, d1 = d[:w, :], d[w:2 * w, :]
        i0, i1 = i[:w, :], i[w:2 * w, :]
        take = d1 < d0
        d = jnp.where(take, d1, d0)
        i = jnp.where(take, i1, i0)
        w //= 2
    idx_row = i

    k_col = jax.lax.broadcasted_iota(jnp.int32, (K, S), 0)
    onehot_t = (k_col == idx_row).astype(jnp.float32)
    quantized = jax.lax.dot_general(onehot_t, emb,
                                    (((0,), (0,)), ((), ())))

    k_vals8 = jnp.broadcast_to(
        jax.lax.broadcasted_iota(jnp.int32, (K, 1), 0).astype(jnp.float32),
        (K, 8))
    idxc = jax.lax.dot_general(onehot_t, k_vals8,
                               (((0,), (0,)), ((), ()))
                               )[:, 0:1].astype(jnp.int32)
    idx_ref[0] = jnp.broadcast_to(idxc, (S, SPAN))

    q_ref[0] = jnp.broadcast_to(quantized[:, None, :], (S, SPAN, D))

    diff2 = (quantized - pooled) ** 2
    ones_8s = jnp.full((8, S), 1.0, jnp.float32)
    row8 = jax.lax.dot_general(ones_8s, diff2,
                               (((1,), (0,)), ((), ())),
                               precision=_HI)
    ones_d8b = jnp.full((D, 8), 1.0, jnp.float32)
    sq11 = jax.lax.dot_general(row8, ones_d8b,
                               (((1,), (0,)), ((), ())),
                               precision=_HI)[0:1, 0:1]
    ones_8t = jnp.full((8, T), 1.0, jnp.float32)
    bound8 = jnp.broadcast_to(bound, (T, 8))
    sb11 = jax.lax.dot_general(ones_8t, bound8,
                               (((1,), (0,)), ((), ())),
                               precision=_HI)[0:1, 0:1]

    @pl.when(b == 0)
    def _():
        acc_ref[:, 0:1] = sq11
        acc_ref[:, 1:2] = sb11

    @pl.when(b > 0)
    def _():
        acc_ref[:, 0:1] += sq11
        acc_ref[:, 1:2] += sb11

    @pl.when(b == B - 1)
    def _():
        e_latent = acc_ref[:, 0:1] * (1.0 / (B * S * D))
        rate = acc_ref[:, 1:2] * (1.0 / (B * T))
        loss_ref[...] = 0.25 * e_latent + 0.01 * (rate - 1.0 / SPAN) ** 2


@jax.jit
def kernel(x, embedding, Wb, bb):
    x4 = x.reshape(B, S, SPAN, D)
    wb2 = Wb.reshape(1, D)
    bb2 = jnp.asarray(bb, jnp.float32).reshape(1, 1)

    q4, idx3, bnd, loss = pl.pallas_call(
        _vq_body,
        grid=(B,),
        in_specs=[
            pl.BlockSpec((1, S, SPAN, D), lambda b: (b, 0, 0, 0)),
            pl.BlockSpec((K, D), lambda b: (0, 0)),
            pl.BlockSpec((1, D), lambda b: (0, 0)),
            pl.BlockSpec((1, 1), lambda b: (0, 0), memory_space=pltpu.SMEM),
        ],
        out_specs=[
            pl.BlockSpec((1, S, SPAN, D), lambda b: (b, 0, 0, 0)),
            pl.BlockSpec((1, S, SPAN), lambda b: (b, 0, 0)),
            pl.BlockSpec((1, T, 1), lambda b: (b, 0, 0)),
            pl.BlockSpec((1, 1), lambda b: (0, 0)),
        ],
        out_shape=[
            jax.ShapeDtypeStruct((B, S, SPAN, D), jnp.float32),
            jax.ShapeDtypeStruct((B, S, SPAN), jnp.int32),
            jax.ShapeDtypeStruct((B, T, 1), jnp.float32),
            jax.ShapeDtypeStruct((1, 1), jnp.float32),
        ],
        scratch_shapes=[pltpu.VMEM((1, 2), jnp.float32),
                        pltpu.VMEM((K, 8), jnp.float32)],
    )(x4, embedding, wb2, bb2)

    quantized_out = q4.reshape(B, T, D)
    indices_out = idx3.reshape(B, T)
    total_loss = loss[0, 0]
    return quantized_out, total_loss, indices_out, bnd.reshape(B, T)


# speedup vs baseline: 2.1666x; 1.0802x over previous
import jax
import jax.numpy as jnp
from jax.experimental import pallas as pl
from jax.experimental.pallas import tpu as pltpu

B, T, D = 8, 2048, 256
K = 1024
SPAN = 8
S = T // SPAN

_HI = jax.lax.Precision.HIGHEST


def _en_body(emb_ref, en_ref):
    ones_d8 = jnp.full((D, 8), 1.0, jnp.float32)
    emb = emb_ref[...]
    en_ref[...] = jax.lax.dot_general(emb * emb, ones_d8,
                                      (((1,), (0,)), ((), ())),
                                      precision=_HI)


def _vq_body(x_ref, emb_ref, wb_ref, bb_ref, en8_ref,
             q_ref, idx_ref, bnd_ref, loss_ref, acc_ref):
    b = pl.program_id(0)

    x4 = x_ref[0]
    xb = x4.reshape(T, D)
    wb = wb_ref[...]
    bb = bb_ref[0, 0]
    emb = emb_ref[...]

    wb8 = jnp.broadcast_to(wb, (8, D))
    logits8 = jax.lax.dot_general(xb, wb8,
                                  (((1,), (1,)), ((), ()))) + bb
    bound = (logits8[:, 0:1] > 0.0).astype(jnp.float32)
    bnd_ref[0] = bound

    acc = x4[:, 0, :]
    for j in range(1, SPAN):
        acc = acc + x4[:, j, :]
    pooled = acc * (1.0 / SPAN)

    en_col = en8_ref[:, 0:1]

    dots_t = jax.lax.dot_general(emb, pooled,
                                 (((1,), (1,)), ((), ())))
    dist_t = en_col - 2.0 * dots_t

    d = dist_t
    i = jax.lax.broadcasted_iota(jnp.int32, (K, S), 0)
    w = K // 2
    while w >= 1:
        d0, d1 = d[:w, :], d[w:2 * w, :]
        i0, i1 = i[:w, :], i[w:2 * w, :]
        take = d1 < d0
        d = jnp.where(take, d1, d0)
        i = jnp.where(take, i1, i0)
        w //= 2
    idx_row = i

    k_col = jax.lax.broadcasted_iota(jnp.int32, (K, S), 0)
    onehot_t = (k_col == idx_row).astype(jnp.float32)
    quantized = jax.lax.dot_general(onehot_t, emb,
                                    (((0,), (0,)), ((), ())))

    k_vals8 = jnp.broadcast_to(
        jax.lax.broadcasted_iota(jnp.int32, (K, 1), 0).astype(jnp.float32),
        (K, 8))
    idxc = jax.lax.dot_general(onehot_t, k_vals8,
                               (((0,), (0,)), ((), ()))
                               )[:, 0:1].astype(jnp.int32)
    idx_ref[0] = jnp.broadcast_to(idxc, (S, SPAN))

    q_ref[0] = jnp.broadcast_to(quantized[:, None, :], (S, SPAN, D))

    diff2 = (quantized - pooled) ** 2
    ones_8s = jnp.full((8, S), 1.0, jnp.float32)
    row8 = jax.lax.dot_general(ones_8s, diff2,
                               (((1,), (0,)), ((), ())))
    ones_d8b = jnp.full((D, 8), 1.0, jnp.float32)
    sq11 = jax.lax.dot_general(row8, ones_d8b,
                               (((1,), (0,)), ((), ()))
                               )[0:1, 0:1]
    ones_8t = jnp.full((8, T), 1.0, jnp.float32)
    bound8 = jnp.broadcast_to(bound, (T, 8))
    sb11 = jax.lax.dot_general(ones_8t, bound8,
                               (((1,), (0,)), ((), ()))
                               )[0:1, 0:1]

    @pl.when(b == 0)
    def _():
        acc_ref[:, 0:1] = sq11
        acc_ref[:, 1:2] = sb11

    @pl.when(b > 0)
    def _():
        acc_ref[:, 0:1] += sq11
        acc_ref[:, 1:2] += sb11

    @pl.when(b == B - 1)
    def _():
        e_latent = acc_ref[:, 0:1] * (1.0 / (B * S * D))
        rate = acc_ref[:, 1:2] * (1.0 / (B * T))
        loss_ref[...] = 0.25 * e_latent + 0.01 * (rate - 1.0 / SPAN) ** 2


@jax.jit
def kernel(x, embedding, Wb, bb):
    x4 = x.reshape(B, S, SPAN, D)
    wb2 = Wb.reshape(1, D)
    bb2 = jnp.asarray(bb, jnp.float32).reshape(1, 1)

    en8 = pl.pallas_call(
        _en_body,
        out_shape=jax.ShapeDtypeStruct((K, 8), jnp.float32),
    )(embedding)

    q4, idx3, bnd, loss = pl.pallas_call(
        _vq_body,
        grid=(B,),
        in_specs=[
            pl.BlockSpec((1, S, SPAN, D), lambda b: (b, 0, 0, 0)),
            pl.BlockSpec((K, D), lambda b: (0, 0)),
            pl.BlockSpec((1, D), lambda b: (0, 0)),
            pl.BlockSpec((1, 1), lambda b: (0, 0), memory_space=pltpu.SMEM),
            pl.BlockSpec((K, 8), lambda b: (0, 0)),
        ],
        out_specs=[
            pl.BlockSpec((1, S, SPAN, D), lambda b: (b, 0, 0, 0)),
            pl.BlockSpec((1, S, SPAN), lambda b: (b, 0, 0)),
            pl.BlockSpec((1, T, 1), lambda b: (b, 0, 0)),
            pl.BlockSpec((1, 1), lambda b: (0, 0)),
        ],
        out_shape=[
            jax.ShapeDtypeStruct((B, S, SPAN, D), jnp.float32),
            jax.ShapeDtypeStruct((B, S, SPAN), jnp.int32),
            jax.ShapeDtypeStruct((B, T, 1), jnp.float32),
            jax.ShapeDtypeStruct((1, 1), jnp.float32),
        ],
        scratch_shapes=[pltpu.VMEM((1, 2), jnp.float32)],
    )(x4, embedding, wb2, bb2, en8)

    quantized_out = q4.reshape(B, T, D)
    indices_out = idx3.reshape(B, T)
    total_loss = loss[0, 0]
    return quantized_out, total_loss, indices_out, bnd.reshape(B, T)
